# baseline (speedup 1.0000x reference)
import functools

import jax
import jax.numpy as jnp
from jax import lax
from jax.experimental import pallas as pl
from jax.experimental.pallas import tpu as pltpu
from jax.experimental.pallas import tpu_sc as plsc

_LANES = 16

_mesh = plsc.VectorSubcoreMesh(core_axis_name="c", subcore_axis_name="s")


@functools.partial(
    pl.kernel,
    mesh=_mesh,
    out_type=jax.ShapeDtypeStruct((_LANES,), jnp.float32),
    scratch_types=[
        pltpu.VMEM((_LANES,), jnp.float32),
        pltpu.VMEM((_LANES,), jnp.float32),
        pltpu.VMEM((_LANES,), jnp.float32),
    ],
)
def _masked_sse(y_hbm, x_hbm, out_hbm, y_v, x_v, o_v):
    cid = lax.axis_index("c")
    sid = lax.axis_index("s")

    @pl.when(jnp.logical_and(cid == 0, sid == 0))
    def _():
        pltpu.sync_copy(y_hbm, y_v)
        pltpu.sync_copy(x_hbm, x_v)
        yv = y_v[...]
        xv = x_v[...]
        d = jnp.where(yv != yv, 0.0, xv - yv)
        v = d * d
        s = v[0] + v[1]
        o_v[...] = jnp.full((_LANES,), s, dtype=jnp.float32)
        pltpu.sync_copy(o_v, out_hbm)


def kernel(y, x):
    pad = jnp.zeros((_LANES - 2,), dtype=jnp.float32)
    y16 = jnp.concatenate([y.astype(jnp.float32), pad])
    x16 = jnp.concatenate([x.astype(jnp.float32), pad])
    out = _masked_sse(y16, x16)
    s = out[0]
    return (s, s)


# speedup vs baseline: 1.0536x; 1.0536x over previous
import functools

import jax
import jax.numpy as jnp
from jax import lax
from jax.experimental import pallas as pl
from jax.experimental.pallas import tpu as pltpu
from jax.experimental.pallas import tpu_sc as plsc

_LANES = 16

_mesh = plsc.VectorSubcoreMesh(core_axis_name="c", subcore_axis_name="s", num_cores=1)


@functools.partial(
    pl.kernel,
    mesh=_mesh,
    out_type=jax.ShapeDtypeStruct((_LANES,), jnp.float32),
    scratch_types=[
        pltpu.VMEM((_LANES,), jnp.float32),
        pltpu.VMEM((_LANES,), jnp.float32),
        pltpu.VMEM((_LANES,), jnp.float32),
    ],
)
def _masked_sse(y_hbm, x_hbm, out_hbm, y_v, x_v, o_v):
    cid = lax.axis_index("c")
    sid = lax.axis_index("s")

    @pl.when(jnp.logical_and(cid == 0, sid == 0))
    def _():
        pltpu.sync_copy(y_hbm, y_v)
        pltpu.sync_copy(x_hbm, x_v)
        yv = y_v[...]
        xv = x_v[...]
        d = jnp.where(yv != yv, 0.0, xv - yv)
        v = d * d
        s = v[0] + v[1]
        o_v[...] = jnp.full((_LANES,), s, dtype=jnp.float32)
        pltpu.sync_copy(o_v, out_hbm)


def kernel(y, x):
    pad = jnp.zeros((_LANES - 2,), dtype=jnp.float32)
    y16 = jnp.concatenate([y.astype(jnp.float32), pad])
    x16 = jnp.concatenate([x.astype(jnp.float32), pad])
    out = _masked_sse(y16, x16)
    s = out[0]
    return (s, s)
